# Initial kernel scaffold; baseline (speedup 1.0000x reference)
import jax
import jax.numpy as jnp
from jax.experimental import pallas as pl


def kernel(seq, adj, W, b, prelu_w):
    raise NotImplementedError("write your pallas kernel here")


# speedup vs baseline: 13.6245x; 13.6245x over previous
import functools

import jax
import jax.numpy as jnp
from jax import lax
from jax.experimental import pallas as pl
from jax.experimental.pallas import tpu as pltpu
from jax.experimental.pallas import tpu_sc as plsc

_NC, _NS, _L = 2, 16, 16
_NW = _NC * _NS
_CHUNK = 128
_D = 128

_f32 = jnp.float32
_i32 = jnp.int32


def _mesh():
    return plsc.VectorSubcoreMesh(
        core_axis_name="c", subcore_axis_name="s",
        num_cores=_NC, num_subcores=_NS)


def _deg_sc(col_p, npad):
    e_pad = col_p.shape[0]
    epw = e_pad // _NW
    steps = epw // _CHUNK
    rpt = npad // _NS

    @functools.partial(
        pl.kernel,
        out_type=jax.ShapeDtypeStruct((_NC * npad,), _f32),
        mesh=_mesh(),
        scratch_types=[
            pltpu.VMEM((_CHUNK,), _i32),
            pltpu.VMEM((_CHUNK,), _f32),
            pltpu.VMEM((rpt,), _f32),
            pltpu.VMEM_SHARED((npad,), _f32),
        ],
    )
    def k(col_hbm, deg_hbm, cidx, ones_v, zbuf, deg_sh):
        cid = lax.axis_index("c")
        sid = lax.axis_index("s")

        def fill_ones(i, c):
            ones_v[pl.ds(i * _L, _L)] = jnp.ones((_L,), _f32)
            return c
        lax.fori_loop(0, _CHUNK // _L, fill_ones, 0)

        def fill_zeros(i, c):
            zbuf[pl.ds(i * _L, _L)] = jnp.zeros((_L,), _f32)
            return c
        lax.fori_loop(0, rpt // _L, fill_zeros, 0)
        pltpu.sync_copy(zbuf, deg_sh.at[pl.ds(sid * rpt, rpt)])
        plsc.subcore_barrier()

        base = (sid * _NC + cid) * epw

        def step(i, c):
            pltpu.sync_copy(col_hbm.at[pl.ds(base + i * _CHUNK, _CHUNK)], cidx)
            pltpu.sync_copy(ones_v, deg_sh.at[cidx], add=True)
            return c
        lax.fori_loop(0, steps, step, 0)
        plsc.subcore_barrier()

        pltpu.sync_copy(deg_sh.at[pl.ds(sid * rpt, rpt)],
                        deg_hbm.at[pl.ds(cid * npad + sid * rpt, rpt)])

    return k(col_p)


def _linear_tc(seq, W, dega, degb):
    n = seq.shape[0]
    blk = 1000

    def body(seq_ref, w_ref, da_ref, db_ref, y_ref, dis_ref):
        deg = da_ref[...] + db_ref[...] + 1.0
        dis = lax.rsqrt(deg)
        x = lax.dot_general(seq_ref[...], w_ref[...],
                            (((1,), (1,)), ((), ())),
                            preferred_element_type=_f32)
        y_ref[...] = x * dis
        dis_ref[...] = dis

    return pl.pallas_call(
        body,
        grid=(n // blk,),
        in_specs=[
            pl.BlockSpec((blk, _D), lambda i: (i, 0)),
            pl.BlockSpec((_D, _D), lambda i: (0, 0)),
            pl.BlockSpec((blk, 1), lambda i: (i, 0)),
            pl.BlockSpec((blk, 1), lambda i: (i, 0)),
        ],
        out_specs=[
            pl.BlockSpec((blk, _D), lambda i: (i, 0)),
            pl.BlockSpec((blk, 1), lambda i: (i, 0)),
        ],
        out_shape=[
            jax.ShapeDtypeStruct((n, _D), _f32),
            jax.ShapeDtypeStruct((n, 1), _f32),
        ],
    )(seq, W, dega, degb)


def _scatter_sc(y, row_p, col_p, npad):
    e_pad = row_p.shape[0]
    epw = e_pad // _NW
    steps = epw // _CHUNK
    rpt = npad // _NS
    zrows = 64

    @functools.partial(
        pl.kernel,
        out_type=jax.ShapeDtypeStruct((_NC * npad, _D), _f32),
        mesh=_mesh(),
        scratch_types=[
            pltpu.VMEM((_CHUNK,), _i32),
            pltpu.VMEM((_CHUNK,), _i32),
            pltpu.VMEM((_CHUNK, _D), _f32),
            pltpu.VMEM((zrows, _D), _f32),
            pltpu.VMEM_SHARED((npad, _D), _f32),
            pltpu.SemaphoreType.DMA,
        ],
    )
    def k(y_hbm, row_hbm, col_hbm, out_hbm, ridx, cidx, rows_v, zbuf, acc, sem):
        cid = lax.axis_index("c")
        sid = lax.axis_index("s")

        def zrow(r, c):
            def zlane(j, c2):
                zbuf[r, pl.ds(j * _L, _L)] = jnp.zeros((_L,), _f32)
                return c2
            return lax.fori_loop(0, _D // _L, zlane, c)
        lax.fori_loop(0, zrows, zrow, 0)

        def zacc(i, c):
            pltpu.sync_copy(zbuf, acc.at[pl.ds(sid * rpt + i * zrows, zrows)])
            return c
        lax.fori_loop(0, rpt // zrows, zacc, 0)
        plsc.subcore_barrier()

        base = (sid * _NC + cid) * epw

        def step(i, c):
            e = base + i * _CHUNK
            pltpu.sync_copy(row_hbm.at[pl.ds(e, _CHUNK)], ridx)
            pltpu.sync_copy(col_hbm.at[pl.ds(e, _CHUNK)], cidx)
            pltpu.async_copy(y_hbm.at[ridx], rows_v, sem).wait()
            pltpu.sync_copy(rows_v, acc.at[cidx], add=True)
            return c
        lax.fori_loop(0, steps, step, 0)
        plsc.subcore_barrier()

        pltpu.sync_copy(acc.at[pl.ds(sid * rpt, rpt)],
                        out_hbm.at[pl.ds(cid * npad + sid * rpt, rpt)])

    return k(y, row_p, col_p)


def _finish_tc(acc0, acc1, y, dis, b2, pw2):
    n = y.shape[0]
    blk = 1000

    def body(a0_ref, a1_ref, y_ref, dis_ref, b_ref, pw_ref, out_ref):
        s = dis_ref[...] * (a0_ref[...] + a1_ref[...] + y_ref[...]) + b_ref[...]
        out_ref[...] = jnp.where(s >= 0, s, pw_ref[...] * s)

    return pl.pallas_call(
        body,
        grid=(n // blk,),
        in_specs=[
            pl.BlockSpec((blk, _D), lambda i: (i, 0)),
            pl.BlockSpec((blk, _D), lambda i: (i, 0)),
            pl.BlockSpec((blk, _D), lambda i: (i, 0)),
            pl.BlockSpec((blk, 1), lambda i: (i, 0)),
            pl.BlockSpec((1, _D), lambda i: (0, 0)),
            pl.BlockSpec((1, 1), lambda i: (0, 0)),
        ],
        out_specs=pl.BlockSpec((blk, _D), lambda i: (i, 0)),
        out_shape=jax.ShapeDtypeStruct((n, _D), _f32),
    )(acc0, acc1, y, dis, b2, pw2)


def kernel(seq, adj, W, b, prelu_w):
    n = seq.shape[0]
    row = adj[0].astype(_i32)
    col = adj[1].astype(_i32)
    e = row.shape[0]

    npad = ((n + _NS * _L - 1) // (_NS * _L)) * (_NS * _L)
    epad = (-e) % (_NW * _CHUNK)
    row_p = jnp.concatenate([row, jnp.zeros((epad,), _i32)])
    col_p = jnp.concatenate([col, jnp.full((epad,), n, _i32)])

    deg2 = _deg_sc(col_p, npad)
    dega = deg2[:n, None]
    degb = deg2[npad:npad + n, None]

    y, dis = _linear_tc(seq, W, dega, degb)

    accs = _scatter_sc(y, row_p, col_p, npad)

    return _finish_tc(accs[:n], accs[npad:npad + n], y, dis,
                      b.reshape(1, _D), prelu_w.reshape(1, 1))
